# baseline (speedup 1.0000x reference)
import functools

import jax
import jax.numpy as jnp
from jax import lax
from jax.experimental import pallas as pl
from jax.experimental.pallas import tpu as pltpu
from jax.experimental.pallas import tpu_sc as plsc

N = 10000
E = 160000
D = 256
OUT = 4

NC, NS, LANES = 2, 16, 16
NW = NC * NS
NPAD = 10240
NR = NW // 2
NPT = NPAD // NR
DUMMY = NPT
CHUNK = 1600
EH = E // 2
NCHUNKS = EH // CHUNK
CAP = EH + 2 * CHUNK
K = 64
IDXCAP = 12288
NEG_INF = float("-inf")

_mesh = plsc.VectorSubcoreMesh(core_axis_name="c", subcore_axis_name="s")


def _wid():
    return lax.axis_index("s") * NC + lax.axis_index("c")


@functools.partial(
    pl.kernel,
    out_type=(
        jax.ShapeDtypeStruct((NW * CAP,), jnp.int32),
        jax.ShapeDtypeStruct((NW * CAP,), jnp.int32),
        jax.ShapeDtypeStruct((NW, 16), jnp.int32),
    ),
    mesh=_mesh,
    compiler_params=pltpu.CompilerParams(needs_layout_passes=False),
    scratch_types=[
        pltpu.VMEM((CHUNK,), jnp.int32),
        pltpu.VMEM((CHUNK,), jnp.int32),
        pltpu.VMEM((CHUNK,), jnp.int32),
        pltpu.VMEM((CHUNK,), jnp.int32),
        pltpu.VMEM((CHUNK,), jnp.int32),
        pltpu.VMEM((CHUNK,), jnp.int32),
        pltpu.VMEM((CHUNK,), jnp.int32),
        pltpu.VMEM((CHUNK,), jnp.int32),
        pltpu.VMEM((16,), jnp.int32),
        pltpu.SemaphoreType.DMA,
        pltpu.SemaphoreType.DMA,
        pltpu.SemaphoreType.DMA,
        pltpu.SemaphoreType.DMA,
        pltpu.SemaphoreType.DMA,
        pltpu.SemaphoreType.DMA,
        pltpu.SemaphoreType.DMA,
        pltpu.SemaphoreType.DMA,
    ],
)
def _partition_edges(src_hbm, dst_hbm, srcl_hbm, dstl_hbm, meta_hbm,
                     sb0, db0, sb1, db1, cs0, cd0, cs1, cd1, metab,
                     is0, id0, is1, id1, os0, od0, os1, od1):
    wid = _wid()
    lo = lax.shift_right_logical(wid, 1) * NPT
    ebase = (wid & 1) * EH
    base = wid * CAP
    sbufs = (sb0, sb1)
    dbufs = (db0, db1)
    css = (cs0, cs1)
    cds = (cd0, cd1)
    isems = ((is0, id0), (is1, id1))
    osems = ((os0, od0), (os1, od1))
    zero16 = jnp.zeros((LANES,), jnp.int32)
    dummy16 = jnp.full((LANES,), DUMMY, jnp.int32)

    def fill_sentinel(i, carry):
        cs0[pl.ds(i * LANES, LANES)] = zero16
        cd0[pl.ds(i * LANES, LANES)] = dummy16
        cs1[pl.ds(i * LANES, LANES)] = zero16
        cd1[pl.ds(i * LANES, LANES)] = dummy16
        return carry

    lax.fori_loop(0, CHUNK // LANES, fill_sentinel, 0)

    def load(g, b):
        moff = pl.multiple_of(ebase + g * CHUNK, 8)
        pltpu.async_copy(src_hbm.at[pl.ds(moff, CHUNK)], sbufs[b], isems[b][0])
        pltpu.async_copy(dst_hbm.at[pl.ds(moff, CHUNK)], dbufs[b], isems[b][1])

    def wait_load(b):
        pltpu.make_async_copy(src_hbm.at[pl.ds(0, CHUNK)], sbufs[b], isems[b][0]).wait()
        pltpu.make_async_copy(dst_hbm.at[pl.ds(0, CHUNK)], dbufs[b], isems[b][1]).wait()

    def wait_flush(b):
        pltpu.make_async_copy(css[b], srcl_hbm.at[pl.ds(0, CHUNK)], osems[b][0]).wait()
        pltpu.make_async_copy(cds[b], dstl_hbm.at[pl.ds(0, CHUNK)], osems[b][1]).wait()

    load(0, 0)
    load(1, 1)

    def outer(go, ptr):
        for b in range(2):
            g = go * 2 + b
            wait_load(b)

            last = jnp.full((LANES,), LANES - 1, jnp.int32)

            def vec_body(i, nvec):
                for u in range(4):
                    iv = i * 4 + u
                    d = dbufs[b][pl.ds(iv * LANES, LANES)]
                    s = sbufs[b][pl.ds(iv * LANES, LANES)]
                    m = (d >= lo) & (d < lo + NPT)
                    run = plsc.cumsum(m.astype(jnp.int32))
                    tot = nvec + run
                    pos = tot - 1
                    plsc.store_scatter(cds[b], [pos], d - lo, mask=m)
                    plsc.store_scatter(css[b], [pos], s, mask=m)
                    nvec = tot.at[last].get(mode="promise_in_bounds")
                return nvec

            nvec = lax.fori_loop(0, CHUNK // LANES // 4, vec_body,
                                 jnp.zeros((LANES,), jnp.int32))
            n = nvec[0]
            @pl.when(g >= 1)
            def _():
                wait_flush(1 - b)

            ptr = pl.multiple_of(ptr, 8)
            off = pl.multiple_of(base + ptr, 8)
            pltpu.async_copy(css[b], srcl_hbm.at[pl.ds(off, CHUNK)], osems[b][0])
            pltpu.async_copy(cds[b], dstl_hbm.at[pl.ds(off, CHUNK)], osems[b][1])

            @pl.when(g + 2 < NCHUNKS)
            def _():
                load(g + 2, b)

            ptr = ptr + ((n + 7) & -8)
        return ptr

    ptr = lax.fori_loop(0, NCHUNKS // 2, outer, jnp.int32(0))
    wait_flush(1)

    def refill(i, carry):
        cs0[pl.ds(i * LANES, LANES)] = zero16
        cd0[pl.ds(i * LANES, LANES)] = dummy16
        return carry

    lax.fori_loop(0, CHUNK // LANES, refill, 0)
    off = pl.multiple_of(base + pl.multiple_of(ptr, 8), 8)
    pltpu.sync_copy(cs0, srcl_hbm.at[pl.ds(off, CHUNK)])
    pltpu.sync_copy(cd0, dstl_hbm.at[pl.ds(off, CHUNK)])
    metab[...] = jnp.broadcast_to(ptr, (16,))
    pltpu.sync_copy(metab, meta_hbm.at[wid])


_IDX_SLICE = 1024
DP = D // 2
NEGPACK = -8323200


@functools.partial(
    pl.kernel,
    out_type=jax.ShapeDtypeStruct((2 * NPAD, DP), jnp.int32),
    mesh=_mesh,
    compiler_params=pltpu.CompilerParams(needs_layout_passes=False),
    scratch_types=[
        pltpu.VMEM((NPT + 1, DP), jnp.int32),
        pltpu.VMEM((K, DP), jnp.int32),
        pltpu.VMEM((K, DP), jnp.int32),
        pltpu.VMEM((IDXCAP,), jnp.int32),
        pltpu.VMEM((IDXCAP,), jnp.int32),
        pltpu.VMEM((16,), jnp.int32),
        pltpu.SemaphoreType.DMA,
        pltpu.SemaphoreType.DMA,
        pltpu.SemaphoreType.DMA,
        pltpu.SemaphoreType.DMA,
    ],
)
def _segmax(h_hbm, srcl_hbm, dstl_hbm, meta_hbm, agg_hbm,
            acc, rows0, rows1, svb, dvb, metab, si, di, rs0, rs1):
    wid = _wid()
    base = wid * CAP
    pltpu.sync_copy(meta_hbm.at[wid], metab)
    cnt = metab[pl.ds(0, 16)][0]
    nch = lax.shift_right_logical(cnt + (K - 1), 6)
    fast = cnt <= (IDXCAP - CHUNK)
    rows = (rows0, rows1)
    rsems = (rs0, rs1)
    negb = jnp.full((LANES,), NEGPACK, jnp.int32)

    for k in range(IDXCAP // _IDX_SLICE):
        @pl.when(fast & (k * _IDX_SLICE < cnt + K))
        def _(k=k):
            off = pl.multiple_of(base + k * _IDX_SLICE, 8)
            voff = pl.ds(k * _IDX_SLICE, _IDX_SLICE)
            pltpu.async_copy(srcl_hbm.at[pl.ds(off, _IDX_SLICE)], svb.at[voff], si)
            pltpu.async_copy(dstl_hbm.at[pl.ds(off, _IDX_SLICE)], dvb.at[voff], di)

    def initrow(r, carry):
        for v in range(DP // LANES):
            acc[r, pl.ds(v * LANES, LANES)] = negb
        return carry

    lax.fori_loop(0, NPT + 1, initrow, 0)

    for k in range(IDXCAP // _IDX_SLICE):
        @pl.when(fast & (k * _IDX_SLICE < cnt + K))
        def _(k=k):
            off = pl.ds(0, _IDX_SLICE)
            voff = pl.ds(k * _IDX_SLICE, _IDX_SLICE)
            pltpu.make_async_copy(srcl_hbm.at[off], svb.at[voff], si).wait()
            pltpu.make_async_copy(dstl_hbm.at[off], dvb.at[voff], di).wait()

    def extract(dvec_ref, dbase, jv):
        dvec = dvec_ref[pl.ds(dbase + jv * LANES, LANES)]
        return [dvec[l] for l in range(LANES)]

    def edge_block(dls, rbuf, jv):
        sls = [pl.ds(v * LANES, LANES) for v in range(DP // LANES)]
        for l in range(LANES):
            dl = dls[l]
            r = jv * LANES + l
            avs = [plsc.bitcast(acc[dl, sl], jnp.bfloat16) for sl in sls]
            rvs = [plsc.bitcast(rbuf[r, sl], jnp.bfloat16) for sl in sls]
            mvs = [jnp.maximum(a_, r_) for a_, r_ in zip(avs, rvs)]
            for sl, m_ in zip(sls, mvs):
                acc[dl, sl] = plsc.bitcast(m_, jnp.int32)

    @pl.when(fast)
    def _():
        def issue(g, b):
            ioff = pl.multiple_of(g * K, 8)
            pltpu.async_copy(h_hbm.at[svb.at[pl.ds(ioff, K)]], rows[b], rsems[b])

        @pl.when(nch > 0)
        def _():
            issue(0, 0)

        def outer(go, carry):
            for b in range(2):
                g = go * 2 + b

                @pl.when(g < nch)
                def _(g=g, b=b):
                    pltpu.make_async_copy(
                        h_hbm.at[svb.at[pl.ds(0, K)]], rows[b], rsems[b]).wait()

                    @pl.when(g + 1 < nch)
                    def _():
                        issue(g + 1, 1 - b)

                    dbase = pl.multiple_of(g * K, 8)
                    dls0 = extract(dvb, dbase, 0)

                    def jv_body(jv, dls):
                        nxt = extract(dvb, dbase,
                                      jnp.minimum(jv + 1, K // LANES - 1))
                        edge_block(dls, rows[b], jv)
                        return tuple(nxt)

                    lax.fori_loop(0, K // LANES, jv_body, tuple(dls0))
            return carry

        lax.fori_loop(0, lax.shift_right_logical(nch + 1, 1), outer, 0)

    @pl.when(jnp.logical_not(fast))
    def _():
        def issue_s(g, b):
            off = pl.multiple_of(base + g * K, 8)
            voff = pl.ds(b * K, K)
            pltpu.sync_copy(srcl_hbm.at[pl.ds(off, K)], svb.at[voff])
            pltpu.sync_copy(dstl_hbm.at[pl.ds(off, K)], dvb.at[voff])
            pltpu.async_copy(h_hbm.at[svb.at[voff]], rows[b], rsems[b])

        @pl.when(nch > 0)
        def _():
            issue_s(0, 0)

        def outer_s(go, carry):
            for b in range(2):
                g = go * 2 + b

                @pl.when(g < nch)
                def _(g=g, b=b):
                    pltpu.make_async_copy(
                        h_hbm.at[svb.at[pl.ds(b * K, K)]], rows[b], rsems[b]).wait()

                    @pl.when(g + 1 < nch)
                    def _():
                        issue_s(g + 1, 1 - b)

                    def jv_body(jv, c):
                        edge_block(extract(dvb, b * K, jv), rows[b], jv)
                        return c

                    lax.fori_loop(0, K // LANES, jv_body, 0)
            return carry

        lax.fori_loop(0, lax.shift_right_logical(nch + 1, 1), outer_s, 0)

    orow = (wid & 1) * NPAD + lax.shift_right_logical(wid, 1) * NPT
    pltpu.sync_copy(acc.at[pl.ds(0, NPT)], agg_hbm.at[pl.ds(orow, NPT)])


def _gin_mlp(h, aggA, aggB, W1, b1, W2, b2, Wo=None, bo=None):
    R = 512
    last = Wo is not None

    def body(*refs):
        if last:
            h_ref, aggA_ref, aggB_ref, w1_ref, b1_ref, w2_ref, b2_ref, wo_ref, bo_ref, out_ref = refs
        else:
            h_ref, aggA_ref, aggB_ref, w1_ref, b1_ref, w2_ref, b2_ref, out_ref, hb_ref = refs

        def unpack(ai):
            au = ai.astype(jnp.uint32)
            alo = lax.bitcast_convert_type(au.astype(jnp.uint16), jnp.bfloat16)
            ahi = lax.bitcast_convert_type(
                (au >> 16).astype(jnp.uint16), jnp.bfloat16)
            return jnp.concatenate(
                [alo.astype(jnp.float32), ahi.astype(jnp.float32)], axis=1)

        a = jnp.maximum(unpack(aggA_ref[...]), unpack(aggB_ref[...]))
        a = jnp.where(a == NEG_INF, 0.0, a)
        z = h_ref[...] + a
        z = jnp.dot(z, w1_ref[...], preferred_element_type=jnp.float32) + b1_ref[...]
        z = jnp.maximum(z, 0.0)
        z = jnp.dot(z, w2_ref[...], preferred_element_type=jnp.float32) + b2_ref[...]
        z = jnp.maximum(z, 0.0)
        if last:
            z = jnp.dot(z, wo_ref[...], preferred_element_type=jnp.float32) + bo_ref[...]
            out_ref[...] = z
        else:
            out_ref[...] = z
            zlo = lax.bitcast_convert_type(
                z[:, :DP].astype(jnp.bfloat16), jnp.uint16).astype(jnp.uint32)
            zhi = lax.bitcast_convert_type(
                z[:, DP:].astype(jnp.bfloat16), jnp.uint16).astype(jnp.uint32)
            hb_ref[...] = (zlo | (zhi << 16)).astype(jnp.int32)

    in_specs = [
        pl.BlockSpec((R, D), lambda i: (i, 0)),
        pl.BlockSpec((R, DP), lambda i: (i, 0)),
        pl.BlockSpec((R, DP), lambda i: (i + NPAD // R, 0)),
        pl.BlockSpec((D, D), lambda i: (0, 0)),
        pl.BlockSpec((1, D), lambda i: (0, 0)),
        pl.BlockSpec((D, D), lambda i: (0, 0)),
        pl.BlockSpec((1, D), lambda i: (0, 0)),
    ]
    args = [h, aggA, aggB, W1, b1.reshape(1, D), W2, b2.reshape(1, D)]
    if last:
        in_specs += [
            pl.BlockSpec((D, 128), lambda i: (0, 0)),
            pl.BlockSpec((1, 128), lambda i: (0, 0)),
        ]
        args += [Wo, bo]
        out_specs = pl.BlockSpec((R, 128), lambda i: (i, 0))
        out_shape = jax.ShapeDtypeStruct((NPAD, 128), jnp.float32)
    else:
        out_specs = (pl.BlockSpec((R, D), lambda i: (i, 0)),
                     pl.BlockSpec((R, DP), lambda i: (i, 0)))
        out_shape = (jax.ShapeDtypeStruct((NPAD, D), jnp.float32),
                     jax.ShapeDtypeStruct((NPAD, DP), jnp.int32))
    return pl.pallas_call(
        body,
        grid=(NPAD // R,),
        in_specs=in_specs,
        out_specs=out_specs,
        out_shape=out_shape,
    )(*args)


def kernel(x, edge_index,
           l0_W1, l0_b1, l0_W2, l0_b2,
           l1_W1, l1_b1, l1_W2, l1_b2,
           l2_W1, l2_b1, l2_W2, l2_b2,
           l3_W1, l3_b1, l3_W2, l3_b2,
           lin_W, lin_b):
    src = edge_index[0]
    dst = edge_index[1]
    srcl, dstl, meta = _partition_edges(src, dst)

    h = jnp.pad(x, ((0, NPAD - N), (0, 0)))
    xlo = lax.bitcast_convert_type(
        h[:, :DP].astype(jnp.bfloat16), jnp.uint16).astype(jnp.uint32)
    xhi = lax.bitcast_convert_type(
        h[:, DP:].astype(jnp.bfloat16), jnp.uint16).astype(jnp.uint32)
    hb = (xlo | (xhi << 16)).astype(jnp.int32)
    layers = [(l0_W1, l0_b1, l0_W2, l0_b2),
              (l1_W1, l1_b1, l1_W2, l1_b2),
              (l2_W1, l2_b1, l2_W2, l2_b2)]
    for (W1, b1, W2, b2) in layers:
        agg = _segmax(hb, srcl, dstl, meta)
        h, hb = _gin_mlp(h, agg, agg, W1, b1, W2, b2)

    Wo = jnp.pad(lin_W, ((0, 0), (0, 128 - OUT)))
    bo = jnp.pad(lin_b, (0, 128 - OUT)).reshape(1, 128)
    agg = _segmax(hb, srcl, dstl, meta)
    logits = _gin_mlp(h, agg, agg, l3_W1, l3_b1, l3_W2, l3_b2, Wo, bo)
    return logits[:N, :OUT]


# speedup vs baseline: 1.0026x; 1.0026x over previous
import functools

import jax
import jax.numpy as jnp
from jax import lax
from jax.experimental import pallas as pl
from jax.experimental.pallas import tpu as pltpu
from jax.experimental.pallas import tpu_sc as plsc

N = 10000
E = 160000
D = 256
OUT = 4

NC, NS, LANES = 2, 16, 16
NW = NC * NS
NPAD = 10240
NR = NW // 2
NPT = NPAD // NR
DUMMY = NPT
CHUNK = 1600
EH = E // 2
NCHUNKS = EH // CHUNK
CAP = EH + 2 * CHUNK
K = 64
IDXCAP = 12288
NEG_INF = float("-inf")

_mesh = plsc.VectorSubcoreMesh(core_axis_name="c", subcore_axis_name="s")


def _wid():
    return lax.axis_index("s") * NC + lax.axis_index("c")


@functools.partial(
    pl.kernel,
    out_type=(
        jax.ShapeDtypeStruct((NW * CAP,), jnp.int32),
        jax.ShapeDtypeStruct((NW * CAP,), jnp.int32),
        jax.ShapeDtypeStruct((NW, 16), jnp.int32),
    ),
    mesh=_mesh,
    compiler_params=pltpu.CompilerParams(needs_layout_passes=False),
    scratch_types=[
        pltpu.VMEM((CHUNK,), jnp.int32),
        pltpu.VMEM((CHUNK,), jnp.int32),
        pltpu.VMEM((CHUNK,), jnp.int32),
        pltpu.VMEM((CHUNK,), jnp.int32),
        pltpu.VMEM((CHUNK,), jnp.int32),
        pltpu.VMEM((CHUNK,), jnp.int32),
        pltpu.VMEM((CHUNK,), jnp.int32),
        pltpu.VMEM((CHUNK,), jnp.int32),
        pltpu.VMEM((16,), jnp.int32),
        pltpu.SemaphoreType.DMA,
        pltpu.SemaphoreType.DMA,
        pltpu.SemaphoreType.DMA,
        pltpu.SemaphoreType.DMA,
        pltpu.SemaphoreType.DMA,
        pltpu.SemaphoreType.DMA,
        pltpu.SemaphoreType.DMA,
        pltpu.SemaphoreType.DMA,
    ],
)
def _partition_edges(src_hbm, dst_hbm, srcl_hbm, dstl_hbm, meta_hbm,
                     sb0, db0, sb1, db1, cs0, cd0, cs1, cd1, metab,
                     is0, id0, is1, id1, os0, od0, os1, od1):
    wid = _wid()
    lo = lax.shift_right_logical(wid, 1) * NPT
    ebase = (wid & 1) * EH
    base = wid * CAP
    sbufs = (sb0, sb1)
    dbufs = (db0, db1)
    css = (cs0, cs1)
    cds = (cd0, cd1)
    isems = ((is0, id0), (is1, id1))
    osems = ((os0, od0), (os1, od1))
    zero16 = jnp.zeros((LANES,), jnp.int32)
    dummy16 = jnp.full((LANES,), DUMMY, jnp.int32)

    def fill_sentinel(i, carry):
        cs0[pl.ds(i * LANES, LANES)] = zero16
        cd0[pl.ds(i * LANES, LANES)] = dummy16
        cs1[pl.ds(i * LANES, LANES)] = zero16
        cd1[pl.ds(i * LANES, LANES)] = dummy16
        return carry

    lax.fori_loop(0, CHUNK // LANES, fill_sentinel, 0)

    def load(g, b):
        moff = pl.multiple_of(ebase + g * CHUNK, 8)
        pltpu.async_copy(src_hbm.at[pl.ds(moff, CHUNK)], sbufs[b], isems[b][0])
        pltpu.async_copy(dst_hbm.at[pl.ds(moff, CHUNK)], dbufs[b], isems[b][1])

    def wait_load(b):
        pltpu.make_async_copy(src_hbm.at[pl.ds(0, CHUNK)], sbufs[b], isems[b][0]).wait()
        pltpu.make_async_copy(dst_hbm.at[pl.ds(0, CHUNK)], dbufs[b], isems[b][1]).wait()

    def wait_flush(b):
        pltpu.make_async_copy(css[b], srcl_hbm.at[pl.ds(0, CHUNK)], osems[b][0]).wait()
        pltpu.make_async_copy(cds[b], dstl_hbm.at[pl.ds(0, CHUNK)], osems[b][1]).wait()

    load(0, 0)
    load(1, 1)

    def outer(go, ptr):
        for b in range(2):
            g = go * 2 + b
            wait_load(b)

            last = jnp.full((LANES,), LANES - 1, jnp.int32)

            def vec_body(i, nvec):
                for u in range(4):
                    iv = i * 4 + u
                    d = dbufs[b][pl.ds(iv * LANES, LANES)]
                    s = sbufs[b][pl.ds(iv * LANES, LANES)]
                    m = (d >= lo) & (d < lo + NPT)
                    run = plsc.cumsum(m.astype(jnp.int32))
                    tot = nvec + run
                    pos = tot - 1
                    plsc.store_scatter(cds[b], [pos], d - lo, mask=m)
                    plsc.store_scatter(css[b], [pos], s, mask=m)
                    nvec = tot.at[last].get(mode="promise_in_bounds")
                return nvec

            nvec = lax.fori_loop(0, CHUNK // LANES // 4, vec_body,
                                 jnp.zeros((LANES,), jnp.int32))
            n = nvec[0]
            @pl.when(g >= 1)
            def _():
                wait_flush(1 - b)

            ptr = pl.multiple_of(ptr, 8)
            off = pl.multiple_of(base + ptr, 8)
            pltpu.async_copy(css[b], srcl_hbm.at[pl.ds(off, CHUNK)], osems[b][0])
            pltpu.async_copy(cds[b], dstl_hbm.at[pl.ds(off, CHUNK)], osems[b][1])

            @pl.when(g + 2 < NCHUNKS)
            def _():
                load(g + 2, b)

            ptr = ptr + ((n + 7) & -8)
        return ptr

    ptr = lax.fori_loop(0, NCHUNKS // 2, outer, jnp.int32(0))
    wait_flush(1)

    def refill(i, carry):
        cs0[pl.ds(i * LANES, LANES)] = zero16
        cd0[pl.ds(i * LANES, LANES)] = dummy16
        return carry

    lax.fori_loop(0, CHUNK // LANES, refill, 0)
    off = pl.multiple_of(base + pl.multiple_of(ptr, 8), 8)
    pltpu.sync_copy(cs0, srcl_hbm.at[pl.ds(off, CHUNK)])
    pltpu.sync_copy(cd0, dstl_hbm.at[pl.ds(off, CHUNK)])
    metab[...] = jnp.broadcast_to(ptr, (16,))
    pltpu.sync_copy(metab, meta_hbm.at[wid])


_IDX_SLICE = 1024
DP = D // 2
NEGPACK = -8323200


@functools.partial(
    pl.kernel,
    out_type=jax.ShapeDtypeStruct((2 * NPAD, DP), jnp.int32),
    mesh=_mesh,
    compiler_params=pltpu.CompilerParams(needs_layout_passes=False),
    scratch_types=[
        pltpu.VMEM((NPT + 1, DP), jnp.int32),
        pltpu.VMEM((K, DP), jnp.int32),
        pltpu.VMEM((K, DP), jnp.int32),
        pltpu.VMEM((IDXCAP,), jnp.int32),
        pltpu.VMEM((IDXCAP,), jnp.int32),
        pltpu.VMEM((16,), jnp.int32),
        pltpu.SemaphoreType.DMA,
        pltpu.SemaphoreType.DMA,
        pltpu.SemaphoreType.DMA,
        pltpu.SemaphoreType.DMA,
    ],
)
def _segmax(h_hbm, srcl_hbm, dstl_hbm, meta_hbm, agg_hbm,
            acc, rows0, rows1, svb, dvb, metab, si, di, rs0, rs1):
    wid = _wid()
    base = wid * CAP
    pltpu.sync_copy(meta_hbm.at[wid], metab)
    cnt = metab[pl.ds(0, 16)][0]
    nch = lax.shift_right_logical(cnt + (K - 1), 6)
    fast = cnt <= (IDXCAP - CHUNK)
    rows = (rows0, rows1)
    rsems = (rs0, rs1)
    negb = jnp.full((LANES,), NEGPACK, jnp.int32)

    for k in range(IDXCAP // _IDX_SLICE):
        @pl.when(fast & (k * _IDX_SLICE < cnt + K))
        def _(k=k):
            off = pl.multiple_of(base + k * _IDX_SLICE, 8)
            voff = pl.ds(k * _IDX_SLICE, _IDX_SLICE)
            pltpu.async_copy(srcl_hbm.at[pl.ds(off, _IDX_SLICE)], svb.at[voff], si)
            pltpu.async_copy(dstl_hbm.at[pl.ds(off, _IDX_SLICE)], dvb.at[voff], di)

    def initrow(r, carry):
        for v in range(DP // LANES):
            acc[r, pl.ds(v * LANES, LANES)] = negb
        return carry

    lax.fori_loop(0, NPT + 1, initrow, 0)

    for k in range(IDXCAP // _IDX_SLICE):
        @pl.when(fast & (k * _IDX_SLICE < cnt + K))
        def _(k=k):
            off = pl.ds(0, _IDX_SLICE)
            voff = pl.ds(k * _IDX_SLICE, _IDX_SLICE)
            pltpu.make_async_copy(srcl_hbm.at[off], svb.at[voff], si).wait()
            pltpu.make_async_copy(dstl_hbm.at[off], dvb.at[voff], di).wait()

    def edge_block(dvec_ref, dbase, rbuf, jv):
        dvec = dvec_ref[pl.ds(dbase + jv * LANES, LANES)]
        dls = [dvec[l] for l in range(LANES)]
        sls = [pl.ds(v * LANES, LANES) for v in range(DP // LANES)]
        for l in range(LANES):
            dl = dls[l]
            r = jv * LANES + l
            avs = [plsc.bitcast(acc[dl, sl], jnp.bfloat16) for sl in sls]
            rvs = [plsc.bitcast(rbuf[r, sl], jnp.bfloat16) for sl in sls]
            mvs = [jnp.maximum(a_, r_) for a_, r_ in zip(avs, rvs)]
            for sl, m_ in zip(sls, mvs):
                acc[dl, sl] = plsc.bitcast(m_, jnp.int32)

    @pl.when(fast)
    def _():
        def issue(g, b):
            ioff = pl.multiple_of(g * K, 8)
            pltpu.async_copy(h_hbm.at[svb.at[pl.ds(ioff, K)]], rows[b], rsems[b])

        @pl.when(nch > 0)
        def _():
            issue(0, 0)

        def outer(go, carry):
            for b in range(2):
                g = go * 2 + b

                @pl.when(g < nch)
                def _(g=g, b=b):
                    pltpu.make_async_copy(
                        h_hbm.at[svb.at[pl.ds(0, K)]], rows[b], rsems[b]).wait()

                    @pl.when(g + 1 < nch)
                    def _():
                        issue(g + 1, 1 - b)

                    def jv_body(jv, c):
                        edge_block(dvb, pl.multiple_of(g * K, 8), rows[b], jv)
                        return c

                    lax.fori_loop(0, K // LANES, jv_body, 0)
            return carry

        lax.fori_loop(0, lax.shift_right_logical(nch + 1, 1), outer, 0)

    @pl.when(jnp.logical_not(fast))
    def _():
        def issue_s(g, b):
            off = pl.multiple_of(base + g * K, 8)
            voff = pl.ds(b * K, K)
            pltpu.sync_copy(srcl_hbm.at[pl.ds(off, K)], svb.at[voff])
            pltpu.sync_copy(dstl_hbm.at[pl.ds(off, K)], dvb.at[voff])
            pltpu.async_copy(h_hbm.at[svb.at[voff]], rows[b], rsems[b])

        @pl.when(nch > 0)
        def _():
            issue_s(0, 0)

        def outer_s(go, carry):
            for b in range(2):
                g = go * 2 + b

                @pl.when(g < nch)
                def _(g=g, b=b):
                    pltpu.make_async_copy(
                        h_hbm.at[svb.at[pl.ds(b * K, K)]], rows[b], rsems[b]).wait()

                    @pl.when(g + 1 < nch)
                    def _():
                        issue_s(g + 1, 1 - b)

                    def jv_body(jv, c):
                        edge_block(dvb, b * K, rows[b], jv)
                        return c

                    lax.fori_loop(0, K // LANES, jv_body, 0)
            return carry

        lax.fori_loop(0, lax.shift_right_logical(nch + 1, 1), outer_s, 0)

    orow = (wid & 1) * NPAD + lax.shift_right_logical(wid, 1) * NPT
    pltpu.sync_copy(acc.at[pl.ds(0, NPT)], agg_hbm.at[pl.ds(orow, NPT)])


def _gin_mlp(h, aggA, aggB, W1, b1, W2, b2, Wo=None, bo=None):
    R = 512
    last = Wo is not None

    def body(*refs):
        if last:
            h_ref, aggA_ref, aggB_ref, w1_ref, b1_ref, w2_ref, b2_ref, wo_ref, bo_ref, out_ref = refs
        else:
            h_ref, aggA_ref, aggB_ref, w1_ref, b1_ref, w2_ref, b2_ref, out_ref, hb_ref = refs

        def unpack(ai):
            au = ai.astype(jnp.uint32)
            alo = lax.bitcast_convert_type(au.astype(jnp.uint16), jnp.bfloat16)
            ahi = lax.bitcast_convert_type(
                (au >> 16).astype(jnp.uint16), jnp.bfloat16)
            return jnp.concatenate(
                [alo.astype(jnp.float32), ahi.astype(jnp.float32)], axis=1)

        a = jnp.maximum(unpack(aggA_ref[...]), unpack(aggB_ref[...]))
        a = jnp.where(a == NEG_INF, 0.0, a)
        z = h_ref[...] + a
        z = jnp.dot(z, w1_ref[...], preferred_element_type=jnp.float32) + b1_ref[...]
        z = jnp.maximum(z, 0.0)
        z = jnp.dot(z, w2_ref[...], preferred_element_type=jnp.float32) + b2_ref[...]
        z = jnp.maximum(z, 0.0)
        if last:
            z = jnp.dot(z, wo_ref[...], preferred_element_type=jnp.float32) + bo_ref[...]
            out_ref[...] = z
        else:
            out_ref[...] = z
            zlo = lax.bitcast_convert_type(
                z[:, :DP].astype(jnp.bfloat16), jnp.uint16).astype(jnp.uint32)
            zhi = lax.bitcast_convert_type(
                z[:, DP:].astype(jnp.bfloat16), jnp.uint16).astype(jnp.uint32)
            hb_ref[...] = (zlo | (zhi << 16)).astype(jnp.int32)

    in_specs = [
        pl.BlockSpec((R, D), lambda i: (i, 0)),
        pl.BlockSpec((R, DP), lambda i: (i, 0)),
        pl.BlockSpec((R, DP), lambda i: (i + NPAD // R, 0)),
        pl.BlockSpec((D, D), lambda i: (0, 0)),
        pl.BlockSpec((1, D), lambda i: (0, 0)),
        pl.BlockSpec((D, D), lambda i: (0, 0)),
        pl.BlockSpec((1, D), lambda i: (0, 0)),
    ]
    args = [h, aggA, aggB, W1, b1.reshape(1, D), W2, b2.reshape(1, D)]
    if last:
        in_specs += [
            pl.BlockSpec((D, 128), lambda i: (0, 0)),
            pl.BlockSpec((1, 128), lambda i: (0, 0)),
        ]
        args += [Wo, bo]
        out_specs = pl.BlockSpec((R, 128), lambda i: (i, 0))
        out_shape = jax.ShapeDtypeStruct((NPAD, 128), jnp.float32)
    else:
        out_specs = (pl.BlockSpec((R, D), lambda i: (i, 0)),
                     pl.BlockSpec((R, DP), lambda i: (i, 0)))
        out_shape = (jax.ShapeDtypeStruct((NPAD, D), jnp.float32),
                     jax.ShapeDtypeStruct((NPAD, DP), jnp.int32))
    return pl.pallas_call(
        body,
        grid=(NPAD // R,),
        in_specs=in_specs,
        out_specs=out_specs,
        out_shape=out_shape,
    )(*args)


def kernel(x, edge_index,
           l0_W1, l0_b1, l0_W2, l0_b2,
           l1_W1, l1_b1, l1_W2, l1_b2,
           l2_W1, l2_b1, l2_W2, l2_b2,
           l3_W1, l3_b1, l3_W2, l3_b2,
           lin_W, lin_b):
    src = edge_index[0]
    dst = edge_index[1]
    srcl, dstl, meta = _partition_edges(src, dst)

    h = jnp.pad(x, ((0, NPAD - N), (0, 0)))
    xlo = lax.bitcast_convert_type(
        h[:, :DP].astype(jnp.bfloat16), jnp.uint16).astype(jnp.uint32)
    xhi = lax.bitcast_convert_type(
        h[:, DP:].astype(jnp.bfloat16), jnp.uint16).astype(jnp.uint32)
    hb = (xlo | (xhi << 16)).astype(jnp.int32)
    layers = [(l0_W1, l0_b1, l0_W2, l0_b2),
              (l1_W1, l1_b1, l1_W2, l1_b2),
              (l2_W1, l2_b1, l2_W2, l2_b2)]
    for (W1, b1, W2, b2) in layers:
        agg = _segmax(hb, srcl, dstl, meta)
        h, hb = _gin_mlp(h, agg, agg, W1, b1, W2, b2)

    Wo = jnp.pad(lin_W, ((0, 0), (0, 128 - OUT)))
    bo = jnp.pad(lin_b, (0, 128 - OUT)).reshape(1, 128)
    agg = _segmax(hb, srcl, dstl, meta)
    logits = _gin_mlp(h, agg, agg, l3_W1, l3_b1, l3_W2, l3_b2, Wo, bo)
    return logits[:N, :OUT]
